# baseline (speedup 1.0000x reference)
import jax
import jax.numpy as jnp
from jax import lax
from jax.experimental import pallas as pl
from jax.experimental.pallas import tpu as pltpu
from jax.experimental.pallas import tpu_sc as plsc

B, L, DIM = 1024, 200, 64
SEG = 3
NC, NS, LANES = 2, 16, 16
NW = NC * NS
N = B * L
PER_W = N // NW
CH = 200
G = PER_W // CH
SUBS = (0, 128)


V = 1000000


def _body(xi_hbm, si_hbm, word_hbm, comb_hbm, out_hbm,
          idx_v, sidx_v, cidx_v, rows_v, crows_v,
          si0, si1, si2, sg0, sg1, so0, so1):
    c = lax.axis_index("c")
    s = lax.axis_index("s")
    wid = s * NC + c
    iota = lax.iota(jnp.int32, LANES)
    sem_i = (si0, si1, si2)
    sem_g = (sg0, sg1)
    sem_o = (so0, so1)

    def stage1(g):
        b = g % 3
        nbase = wid * PER_W + g * CH
        return [
            pltpu.async_copy(xi_hbm.at[pl.ds(nbase, CH)], idx_v.at[b],
                             sem_i[b]),
            pltpu.async_copy(si_hbm.at[pl.ds(nbase, CH)], sidx_v.at[b],
                             sem_i[b]),
        ]

    def stage2(g, idescs):
        bi = g % 3
        b = g % 2
        nbase = wid * PER_W + g * CH
        for d in idescs:
            d.wait()

        for j in range(CH // 16):
            sl = pl.ds(j * 16, 16)
            cidx_v[b, sl] = (lax.rem(iota + (nbase + j * 16), L) * SEG
                             + sidx_v[bi, sl])
        sl = pl.ds(CH - 16, 16)
        cidx_v[b, sl] = (lax.rem(iota + (nbase + CH - 16), L) * SEG
                         + sidx_v[bi, sl])
        descs = []
        for k, st in enumerate(SUBS):
            w = min(128, CH - st)
            ksl = pl.ds(st, w)
            descs.append(pltpu.async_copy(
                word_hbm.at[idx_v.at[bi, ksl]], rows_v.at[b, ksl], sem_g[b]))
            descs.append(pltpu.async_copy(
                comb_hbm.at[cidx_v.at[b, ksl]], crows_v.at[b, ksl], sem_g[b]))
        return descs

    def stage3(g, gdescs):
        b = g % 2
        nbase = wid * PER_W + g * CH
        for d in gdescs:
            d.wait()

        def add(r4, carry):
            for u in range(4):
                r = r4 * 4 + u
                for cc in range(DIM // 16):
                    sl = pl.ds(cc * 16, 16)
                    crows_v[b, r, sl] = rows_v[b, r, sl] + crows_v[b, r, sl]
            return carry
        lax.fori_loop(0, CH // 4, add, 0)
        return pltpu.async_copy(crows_v.at[b], out_hbm.at[wid * G + g],
                                sem_o[b])

    descs_i = {0: stage1(0), 1: stage1(1)}
    descs_g = {0: stage2(0, descs_i[0])}
    descs_o = {}
    for g in range(G):
        if g + 2 < G:
            descs_i[g + 2] = stage1(g + 2)
        if g + 1 < G:
            if g - 1 >= 0:
                descs_o[g - 1].wait()
            descs_g[g + 1] = stage2(g + 1, descs_i[g + 1])
        descs_o[g] = stage3(g, descs_g[g])
    descs_o[G - 2].wait()
    descs_o[G - 1].wait()


def kernel(x, segment_x, word_table, pos_emb, seg_table):
    xf = x.reshape(N).astype(jnp.int32)
    sf = segment_x.reshape(N).astype(jnp.int32)
    comb = (pos_emb[0, :L, :][:, None, :] + seg_table[None, :, :]
            ).reshape(L * SEG, DIM).astype(jnp.float32)
    mesh = plsc.VectorSubcoreMesh(core_axis_name="c", subcore_axis_name="s",
                                  num_cores=NC, num_subcores=NS)
    wt_pad = jnp.concatenate(
        [word_table, jnp.zeros((V, 2 * DIM - DIM), jnp.float32)], axis=1)
    out = pl.kernel(
        _body,
        out_type=jax.ShapeDtypeStruct((B, L, DIM), jnp.float32),
        mesh=mesh,
        scratch_types=[
            pltpu.VMEM((3, CH), jnp.int32),
            pltpu.VMEM((3, CH), jnp.int32),
            pltpu.VMEM((2, CH), jnp.int32),
            pltpu.VMEM((2, CH, 2 * DIM), jnp.float32),
            pltpu.VMEM((2, CH, DIM), jnp.float32),
            pltpu.SemaphoreType.DMA,
            pltpu.SemaphoreType.DMA,
            pltpu.SemaphoreType.DMA,
            pltpu.SemaphoreType.DMA,
            pltpu.SemaphoreType.DMA,
            pltpu.SemaphoreType.DMA,
            pltpu.SemaphoreType.DMA,
        ],
        compiler_params=pltpu.CompilerParams(use_tc_tiling_on_sc=False),
    )(xf, sf, wt_pad, comb)
    return out


# speedup vs baseline: 1.0582x; 1.0582x over previous
import jax
import jax.numpy as jnp
from jax import lax
from jax.experimental import pallas as pl
from jax.experimental.pallas import tpu as pltpu
from jax.experimental.pallas import tpu_sc as plsc

B, L, DIM = 1024, 200, 64
SEG = 3
NC, NS, LANES = 2, 16, 16
NW = NC * NS
N = B * L
PER_W = N // NW
CH = 200
G = PER_W // CH
SUBS = (0, 128)


V = 1000000


def _body(xi_hbm, si_hbm, word_hbm, comb_hbm, out_hbm,
          idx_v, sidx_v, cidx_v, gidx_v, rows_v, crows_v,
          si0, si1, si2, sg0, sg1, so0, so1):
    c = lax.axis_index("c")
    s = lax.axis_index("s")
    wid = s * NC + c
    iota = lax.iota(jnp.int32, LANES)
    sem_i = (si0, si1, si2)
    sem_g = (sg0, sg1)
    sem_o = (so0, so1)

    def stage1(g):
        b = g % 3
        nbase = wid * PER_W + g * CH
        return [
            pltpu.async_copy(xi_hbm.at[pl.ds(nbase, CH)], idx_v.at[b],
                             sem_i[b]),
            pltpu.async_copy(si_hbm.at[pl.ds(nbase, CH)], sidx_v.at[b],
                             sem_i[b]),
        ]

    def stage2(g, idescs):
        bi = g % 3
        b = g % 2
        nbase = wid * PER_W + g * CH
        for d in idescs:
            d.wait()

        for j in range(CH // 16):
            sl = pl.ds(j * 16, 16)
            cidx_v[b, sl] = (lax.rem(iota + (nbase + j * 16), L) * SEG
                             + sidx_v[bi, sl])
            gidx_v[b, sl] = idx_v[bi, sl] * 2
        sl = pl.ds(CH - 16, 16)
        cidx_v[b, sl] = (lax.rem(iota + (nbase + CH - 16), L) * SEG
                         + sidx_v[bi, sl])
        gidx_v[b, sl] = idx_v[bi, sl] * 2
        descs = []
        for k, st in enumerate(SUBS):
            w = min(128, CH - st)
            ksl = pl.ds(st, w)
            descs.append(pltpu.async_copy(
                word_hbm.at[gidx_v.at[b, ksl]], rows_v.at[b, ksl], sem_g[b]))
            descs.append(pltpu.async_copy(
                comb_hbm.at[cidx_v.at[b, ksl]], crows_v.at[b, ksl], sem_g[b]))
        return descs

    def stage3(g, gdescs):
        b = g % 2
        nbase = wid * PER_W + g * CH
        for d in gdescs:
            d.wait()

        def add(r, carry):
            for cc in range(DIM // 16):
                sl = pl.ds(cc * 16, 16)
                crows_v[b, r, sl] = rows_v[b, r, sl] + crows_v[b, r, sl]
            return carry
        lax.fori_loop(0, CH, add, 0)
        return pltpu.async_copy(crows_v.at[b], out_hbm.at[wid * G + g],
                                sem_o[b])

    descs_i = {0: stage1(0), 1: stage1(1)}
    descs_g = {0: stage2(0, descs_i[0])}
    descs_o = {}
    for g in range(G):
        if g + 2 < G:
            descs_i[g + 2] = stage1(g + 2)
        if g + 1 < G:
            if g - 1 >= 0:
                descs_o[g - 1].wait()
            descs_g[g + 1] = stage2(g + 1, descs_i[g + 1])
        descs_o[g] = stage3(g, descs_g[g])
    descs_o[G - 2].wait()
    descs_o[G - 1].wait()


def kernel(x, segment_x, word_table, pos_emb, seg_table):
    xf = x.reshape(N).astype(jnp.int32)
    sf = segment_x.reshape(N).astype(jnp.int32)
    comb = (pos_emb[0, :L, :][:, None, :] + seg_table[None, :, :]
            ).reshape(L * SEG, DIM).astype(jnp.float32)
    mesh = plsc.VectorSubcoreMesh(core_axis_name="c", subcore_axis_name="s",
                                  num_cores=NC, num_subcores=NS)
    wt_pad = jnp.concatenate(
        [word_table, jnp.zeros((V, 2 * DIM - DIM), jnp.float32)], axis=1)
    out = pl.kernel(
        _body,
        out_type=jax.ShapeDtypeStruct((B, L, DIM), jnp.float32),
        mesh=mesh,
        scratch_types=[
            pltpu.VMEM((3, CH), jnp.int32),
            pltpu.VMEM((3, CH), jnp.int32),
            pltpu.VMEM((2, CH), jnp.int32),
            pltpu.VMEM((2, CH), jnp.int32),
            pltpu.VMEM((2, CH, DIM), jnp.float32),
            pltpu.VMEM((2, CH, DIM), jnp.float32),
            pltpu.SemaphoreType.DMA,
            pltpu.SemaphoreType.DMA,
            pltpu.SemaphoreType.DMA,
            pltpu.SemaphoreType.DMA,
            pltpu.SemaphoreType.DMA,
            pltpu.SemaphoreType.DMA,
            pltpu.SemaphoreType.DMA,
        ],
        compiler_params=pltpu.CompilerParams(use_tc_tiling_on_sc=False),
    )(xf, sf, wt_pad.reshape(2 * V, DIM), comb)
    return out
